# baseline (speedup 1.0000x reference)
import numpy as np
import jax
import jax.numpy as jnp
from jax.experimental import pallas as pl

_PARENT = [-1, 0, 0, 0, 0, 1, 1, 2, 3, 4, 4, 5, 5, 6, 7, 8, 9, 10,
           11, 12, 13, 14, 14, 14, 15, 15, 16, 17]
_N = 28

_CHILD = [[] for _ in range(_N)]
for _c, _p in enumerate(_PARENT):
    if _p >= 0:
        _CHILD[_p].append(_c)

_deg = np.ones(_N, np.float64)
for _c, _p in enumerate(_PARENT):
    if _p >= 0:
        _deg[_c] += 1.0
        _deg[_p] += 1.0
_dinv = 1.0 / np.sqrt(_deg)
_A = np.zeros((_N, _N), np.float64)
for _i in range(_N):
    _A[_i, _i] = _dinv[_i] * _dinv[_i]
for _c, _p in enumerate(_PARENT):
    if _p >= 0:
        _A[_p, _c] = _dinv[_p] * _dinv[_c]
        _A[_c, _p] = _dinv[_c] * _dinv[_p]
_R = [float(v) for v in _A.sum(1)]
_ATERMS = [[(j, float(_A[i, j])) for j in range(_N) if _A[i, j] != 0.0]
           for i in range(_N)]

_ANC = np.zeros((_N, _N), np.float32)
for _i in range(1, _N):
    _n = _i
    while _n != 0:
        _ANC[_i, _n] = 1.0
        _n = _PARENT[_n]

_GROUPS = []
for _p in range(_N):
    _ch = _CHILD[_p]
    if len(_ch) > 1:
        assert _ch == list(range(_ch[0], _ch[0] + len(_ch)))
        _GROUPS.append((_ch[0], _ch[0] + len(_ch)))
_MASK = np.zeros((_N, 1), np.float32)
for _a, _b in _GROUPS:
    _MASK[_a:_b] = 1.0


def _body(x_ref, W0_ref, b0_ref, W1_ref, b1_ref, W2_ref, b2_ref,
          W3_ref, b3_ref, w4_ref, b4_ref, anc_ref, mask_ref,
          pp_ref, lg_ref):
    f32 = jnp.float32
    bB = x_ref.shape[0]
    xb = x_ref[...]
    y0T = jax.lax.dot_general(W0_ref[...], xb, (((0,), (1,)), ((), ())),
                              preferred_element_type=f32)
    b0 = b0_ref[...].reshape(-1, 1)
    h = jnp.concatenate(
        [jax.nn.relu(_R[i] * y0T + b0) for i in range(_N)], axis=1)

    for W_ref, b_ref in ((W1_ref, b1_ref), (W2_ref, b2_ref),
                         (W3_ref, b3_ref)):
        g = jax.lax.dot_general(W_ref[...], h, (((0,), (0,)), ((), ())),
                                preferred_element_type=f32)
        b = b_ref[...].reshape(-1, 1)
        h = jnp.concatenate([
            jax.nn.relu(sum(c * g[:, j * bB:(j + 1) * bB]
                            for j, c in _ATERMS[i]) + b)
            for i in range(_N)], axis=1)

    w4 = w4_ref[...]
    z = jnp.sum(h * w4, axis=0, keepdims=True)
    b4 = b4_ref[0, 0]
    lg = jnp.concatenate([
        sum(c * z[:, j * bB:(j + 1) * bB] for j, c in _ATERMS[i])
        for i in range(_N)], axis=0) + b4

    lses = []
    for a, bnd in _GROUPS:
        seg = lg[a:bnd]
        m = jnp.max(seg, axis=0, keepdims=True)
        lse = m + jnp.log(jnp.sum(jnp.exp(seg - m), axis=0, keepdims=True))
        lses.append(jnp.broadcast_to(lse, (bnd - a, bB)))
    zrow = jnp.zeros((1, bB), f32)
    lse_node = jnp.concatenate([
        zrow,
        lses[0],
        lses[1],
        jnp.broadcast_to(zrow, (2, bB)),
        lses[2],
        lses[3],
        jnp.broadcast_to(zrow, (8, bB)),
        lses[4],
        lses[5],
        jnp.broadcast_to(zrow, (2, bB)),
    ], axis=0)
    s = mask_ref[...] * lg - lse_node
    logp = jnp.dot(anc_ref[...], s, preferred_element_type=f32)
    pp_ref[...] = jnp.exp(logp).T
    lg_ref[...] = lg.T


def kernel(x, W0, b0, W1, b1, W2, b2, W3, b3, W4, b4):
    B = x.shape[0]
    bB = 1024
    while B % bB:
        bB //= 2
    args = (x, W0, b0.reshape(1, -1), W1, b1.reshape(1, -1),
            W2, b2.reshape(1, -1), W3, b3.reshape(1, -1),
            W4, b4.reshape(1, 1),
            jnp.asarray(_ANC), jnp.asarray(_MASK))
    in_specs = [pl.BlockSpec((bB, x.shape[1]), lambda i: (i, 0))]
    for a in args[1:]:
        in_specs.append(pl.BlockSpec(a.shape, lambda i: (0, 0)))
    out_specs = [pl.BlockSpec((bB, _N), lambda i: (i, 0))] * 2
    out_shape = [jax.ShapeDtypeStruct((B, _N), x.dtype)] * 2
    pp, lg = pl.pallas_call(
        _body, grid=(B // bB,), in_specs=in_specs,
        out_specs=out_specs, out_shape=out_shape)(*args)
    return pp, lg


# speedup vs baseline: 1.0072x; 1.0072x over previous
import numpy as np
import jax
import jax.numpy as jnp
from jax.experimental import pallas as pl

_PARENT = [-1, 0, 0, 0, 0, 1, 1, 2, 3, 4, 4, 5, 5, 6, 7, 8, 9, 10,
           11, 12, 13, 14, 14, 14, 15, 15, 16, 17]
_N = 28

_CHILD = [[] for _ in range(_N)]
for _c, _p in enumerate(_PARENT):
    if _p >= 0:
        _CHILD[_p].append(_c)

_deg = np.ones(_N, np.float64)
for _c, _p in enumerate(_PARENT):
    if _p >= 0:
        _deg[_c] += 1.0
        _deg[_p] += 1.0
_dinv = 1.0 / np.sqrt(_deg)
_A = np.zeros((_N, _N), np.float64)
for _i in range(_N):
    _A[_i, _i] = _dinv[_i] * _dinv[_i]
for _c, _p in enumerate(_PARENT):
    if _p >= 0:
        _A[_p, _c] = _dinv[_p] * _dinv[_c]
        _A[_c, _p] = _dinv[_c] * _dinv[_p]
_R = [float(v) for v in _A.sum(1)]
_ATERMS = [[(j, float(_A[i, j])) for j in range(_N) if _A[i, j] != 0.0]
           for i in range(_N)]

_ANC = np.zeros((_N, _N), np.float32)
for _i in range(1, _N):
    _n = _i
    while _n != 0:
        _ANC[_i, _n] = 1.0
        _n = _PARENT[_n]

_GROUPS = []
for _p in range(_N):
    _ch = _CHILD[_p]
    if len(_ch) > 1:
        assert _ch == list(range(_ch[0], _ch[0] + len(_ch)))
        _GROUPS.append((_ch[0], _ch[0] + len(_ch)))
_MASK = np.zeros((_N, 1), np.float32)
for _a, _b in _GROUPS:
    _MASK[_a:_b] = 1.0


def _body(x_ref, W0_ref, b0_ref, W1_ref, b1_ref, W2_ref, b2_ref,
          W3_ref, b3_ref, w4_ref, b4_ref, anc_ref, mask_ref,
          pp_ref, lg_ref):
    f32 = jnp.float32
    bB = x_ref.shape[0]
    xb = x_ref[...]
    y0 = jnp.dot(xb, W0_ref[...], preferred_element_type=f32)
    y0T = y0.T
    b0 = b0_ref[...].T
    h = jnp.concatenate(
        [jax.nn.relu(_R[i] * y0T + b0) for i in range(_N)], axis=1)

    for W_ref, b_ref in ((W1_ref, b1_ref), (W2_ref, b2_ref),
                         (W3_ref, b3_ref)):
        g = jnp.dot(W_ref[...].T, h, preferred_element_type=f32)
        b = b_ref[...].T
        h = jnp.concatenate([
            jax.nn.relu(sum(c * g[:, j * bB:(j + 1) * bB]
                            for j, c in _ATERMS[i]) + b)
            for i in range(_N)], axis=1)

    w4 = w4_ref[...]
    z = jnp.sum(h * w4, axis=0, keepdims=True)
    b4 = b4_ref[0, 0]
    lg = jnp.concatenate([
        sum(c * z[:, j * bB:(j + 1) * bB] for j, c in _ATERMS[i])
        for i in range(_N)], axis=0) + b4

    lses = []
    for a, bnd in _GROUPS:
        seg = lg[a:bnd]
        m = jnp.max(seg, axis=0, keepdims=True)
        lse = m + jnp.log(jnp.sum(jnp.exp(seg - m), axis=0, keepdims=True))
        lses.append(jnp.broadcast_to(lse, (bnd - a, bB)))
    zrow = jnp.zeros((1, bB), f32)
    lse_node = jnp.concatenate([
        zrow,
        lses[0],
        lses[1],
        jnp.broadcast_to(zrow, (2, bB)),
        lses[2],
        lses[3],
        jnp.broadcast_to(zrow, (8, bB)),
        lses[4],
        lses[5],
        jnp.broadcast_to(zrow, (2, bB)),
    ], axis=0)
    s = mask_ref[...] * lg - lse_node
    logp = jnp.dot(anc_ref[...], s, preferred_element_type=f32)
    pp_ref[...] = jnp.exp(logp).T
    lg_ref[...] = lg.T


def kernel(x, W0, b0, W1, b1, W2, b2, W3, b3, W4, b4):
    B = x.shape[0]
    bB = 1024
    while B % bB:
        bB //= 2
    args = (x, W0, b0.reshape(1, -1), W1, b1.reshape(1, -1),
            W2, b2.reshape(1, -1), W3, b3.reshape(1, -1),
            W4, b4.reshape(1, 1),
            jnp.asarray(_ANC), jnp.asarray(_MASK))
    in_specs = [pl.BlockSpec((bB, x.shape[1]), lambda i: (i, 0))]
    for a in args[1:]:
        in_specs.append(pl.BlockSpec(a.shape, lambda i: (0, 0)))
    out_specs = [pl.BlockSpec((bB, _N), lambda i: (i, 0))] * 2
    out_shape = [jax.ShapeDtypeStruct((B, _N), x.dtype)] * 2
    pp, lg = pl.pallas_call(
        _body, grid=(B // bB,), in_specs=in_specs,
        out_specs=out_specs, out_shape=out_shape)(*args)
    return pp, lg


# speedup vs baseline: 1.6522x; 1.6404x over previous
import numpy as np
import jax
import jax.numpy as jnp
from jax.experimental import pallas as pl

_PARENT = [-1, 0, 0, 0, 0, 1, 1, 2, 3, 4, 4, 5, 5, 6, 7, 8, 9, 10,
           11, 12, 13, 14, 14, 14, 15, 15, 16, 17]
_N = 28

_CHILD = [[] for _ in range(_N)]
for _c, _p in enumerate(_PARENT):
    if _p >= 0:
        _CHILD[_p].append(_c)

_deg = np.ones(_N, np.float64)
for _c, _p in enumerate(_PARENT):
    if _p >= 0:
        _deg[_c] += 1.0
        _deg[_p] += 1.0
_dinv = 1.0 / np.sqrt(_deg)
_A = np.zeros((_N, _N), np.float64)
for _i in range(_N):
    _A[_i, _i] = _dinv[_i] * _dinv[_i]
for _c, _p in enumerate(_PARENT):
    if _p >= 0:
        _A[_p, _c] = _dinv[_p] * _dinv[_c]
        _A[_c, _p] = _dinv[_c] * _dinv[_p]
_R = [float(v) for v in _A.sum(1)]
_ATERMS = [[(j, float(_A[i, j])) for j in range(_N) if _A[i, j] != 0.0]
           for i in range(_N)]

_ANC = np.zeros((_N, _N), np.float32)
for _i in range(1, _N):
    _n = _i
    while _n != 0:
        _ANC[_i, _n] = 1.0
        _n = _PARENT[_n]

_GROUPS = []
for _p in range(_N):
    _ch = _CHILD[_p]
    if len(_ch) > 1:
        assert _ch == list(range(_ch[0], _ch[0] + len(_ch)))
        _GROUPS.append((_ch[0], _ch[0] + len(_ch)))
_MASK = np.zeros((_N, 1), np.float32)
for _a, _b in _GROUPS:
    _MASK[_a:_b] = 1.0


def _body(x_ref, W0T_ref, b0_ref, W1T_ref, b1_ref, W2T_ref, b2_ref,
          W3T_ref, b3_ref, w4_ref, b4_ref, anc_ref, mask_ref,
          pp_ref, lg_ref):
    f32 = jnp.float32
    bB = x_ref.shape[0]
    xb = x_ref[...]
    y0T = jax.lax.dot_general(W0T_ref[...], xb, (((1,), (1,)), ((), ())),
                              preferred_element_type=f32)
    b0 = b0_ref[...].T
    h = jnp.concatenate(
        [jax.nn.relu(_R[i] * y0T + b0) for i in range(_N)], axis=1)

    for WT_ref, b_ref in ((W1T_ref, b1_ref), (W2T_ref, b2_ref),
                          (W3T_ref, b3_ref)):
        g = jnp.dot(WT_ref[...], h, preferred_element_type=f32)
        b = b_ref[...].T
        h = jnp.concatenate([
            jax.nn.relu(sum(c * g[:, j * bB:(j + 1) * bB]
                            for j, c in _ATERMS[i]) + b)
            for i in range(_N)], axis=1)

    w4 = w4_ref[...].T
    z = jnp.sum(h * w4, axis=0, keepdims=True)
    b4 = b4_ref[0, 0]
    lg = jnp.concatenate([
        sum(c * z[:, j * bB:(j + 1) * bB] for j, c in _ATERMS[i])
        for i in range(_N)], axis=0) + b4

    lses = []
    for a, bnd in _GROUPS:
        seg = lg[a:bnd]
        m = jnp.max(seg, axis=0, keepdims=True)
        lse = m + jnp.log(jnp.sum(jnp.exp(seg - m), axis=0, keepdims=True))
        lses.append(jnp.broadcast_to(lse, (bnd - a, bB)))
    zrow = jnp.zeros((1, bB), f32)
    lse_node = jnp.concatenate([
        zrow,
        lses[0],
        lses[1],
        jnp.broadcast_to(zrow, (2, bB)),
        lses[2],
        lses[3],
        jnp.broadcast_to(zrow, (8, bB)),
        lses[4],
        lses[5],
        jnp.broadcast_to(zrow, (2, bB)),
    ], axis=0)
    s = mask_ref[...] * lg - lse_node
    logp = jnp.dot(anc_ref[...], s, preferred_element_type=f32)
    pp_ref[...] = jnp.exp(logp)
    lg_ref[...] = lg


def kernel(x, W0, b0, W1, b1, W2, b2, W3, b3, W4, b4):
    B = x.shape[0]
    bB = 1024
    while B % bB:
        bB //= 2
    args = (x, W0.T, b0.reshape(1, -1), W1.T, b1.reshape(1, -1),
            W2.T, b2.reshape(1, -1), W3.T, b3.reshape(1, -1),
            W4.reshape(1, -1), b4.reshape(1, 1),
            jnp.asarray(_ANC), jnp.asarray(_MASK))
    in_specs = [pl.BlockSpec((bB, x.shape[1]), lambda i: (i, 0))]
    for a in args[1:]:
        in_specs.append(pl.BlockSpec(a.shape, lambda i: (0, 0)))
    out_specs = [pl.BlockSpec((_N, bB), lambda i: (0, i))] * 2
    out_shape = [jax.ShapeDtypeStruct((_N, B), x.dtype)] * 2
    ppT, lgT = pl.pallas_call(
        _body, grid=(B // bB,), in_specs=in_specs,
        out_specs=out_specs, out_shape=out_shape)(*args)
    return ppT.T, lgT.T


# speedup vs baseline: 1.6774x; 1.0152x over previous
import numpy as np
import jax
import jax.numpy as jnp
from jax.experimental import pallas as pl

_PARENT = [-1, 0, 0, 0, 0, 1, 1, 2, 3, 4, 4, 5, 5, 6, 7, 8, 9, 10,
           11, 12, 13, 14, 14, 14, 15, 15, 16, 17]
_N = 28

_CHILD = [[] for _ in range(_N)]
for _c, _p in enumerate(_PARENT):
    if _p >= 0:
        _CHILD[_p].append(_c)

_deg = np.ones(_N, np.float64)
for _c, _p in enumerate(_PARENT):
    if _p >= 0:
        _deg[_c] += 1.0
        _deg[_p] += 1.0
_dinv = 1.0 / np.sqrt(_deg)
_A = np.zeros((_N, _N), np.float64)
for _i in range(_N):
    _A[_i, _i] = _dinv[_i] * _dinv[_i]
for _c, _p in enumerate(_PARENT):
    if _p >= 0:
        _A[_p, _c] = _dinv[_p] * _dinv[_c]
        _A[_c, _p] = _dinv[_c] * _dinv[_p]
_R = [float(v) for v in _A.sum(1)]
_ATERMS = [[(j, float(_A[i, j])) for j in range(_N) if _A[i, j] != 0.0]
           for i in range(_N)]

_ANC = np.zeros((_N, _N), np.float32)
for _i in range(1, _N):
    _n = _i
    while _n != 0:
        _ANC[_i, _n] = 1.0
        _n = _PARENT[_n]

_GROUPS = []
for _p in range(_N):
    _ch = _CHILD[_p]
    if len(_ch) > 1:
        assert _ch == list(range(_ch[0], _ch[0] + len(_ch)))
        _GROUPS.append((_ch[0], _ch[0] + len(_ch)))
_MASK = np.zeros((_N, 1), np.float32)
for _a, _b in _GROUPS:
    _MASK[_a:_b] = 1.0


def _body(x_ref, W0T_ref, b0_ref, W1T_ref, b1_ref, W2T_ref, b2_ref,
          W3T_ref, b3_ref, w4_ref, b4_ref, anc_ref, mask_ref,
          pp_ref, lg_ref):
    f32 = jnp.float32
    bB = x_ref.shape[0]
    xb = x_ref[...]
    y0T = jax.lax.dot_general(W0T_ref[...], xb, (((1,), (1,)), ((), ())),
                              preferred_element_type=f32)
    b0 = b0_ref[...].T
    h = jnp.concatenate(
        [jax.nn.relu(_R[i] * y0T + b0) for i in range(_N)], axis=1)

    for WT_ref, b_ref in ((W1T_ref, b1_ref), (W2T_ref, b2_ref),
                          (W3T_ref, b3_ref)):
        g = jnp.dot(WT_ref[...], h, preferred_element_type=f32)
        b = b_ref[...].T
        h = jnp.concatenate([
            jax.nn.relu(sum(c * g[:, j * bB:(j + 1) * bB]
                            for j, c in _ATERMS[i]) + b)
            for i in range(_N)], axis=1)

    w4 = w4_ref[...].T
    z = jnp.sum(h * w4, axis=0, keepdims=True)
    b4 = b4_ref[0, 0]
    lg = jnp.concatenate([
        sum(c * z[:, j * bB:(j + 1) * bB] for j, c in _ATERMS[i])
        for i in range(_N)], axis=0) + b4

    lses = []
    for a, bnd in _GROUPS:
        seg = lg[a:bnd]
        m = jnp.max(seg, axis=0, keepdims=True)
        lse = m + jnp.log(jnp.sum(jnp.exp(seg - m), axis=0, keepdims=True))
        lses.append(jnp.broadcast_to(lse, (bnd - a, bB)))
    zrow = jnp.zeros((1, bB), f32)
    lse_node = jnp.concatenate([
        zrow,
        lses[0],
        lses[1],
        jnp.broadcast_to(zrow, (2, bB)),
        lses[2],
        lses[3],
        jnp.broadcast_to(zrow, (8, bB)),
        lses[4],
        lses[5],
        jnp.broadcast_to(zrow, (2, bB)),
    ], axis=0)
    s = mask_ref[...] * lg - lse_node
    logp = jnp.dot(anc_ref[...], s, preferred_element_type=f32)
    pp_ref[...] = jnp.exp(logp)
    lg_ref[...] = lg


def kernel(x, W0, b0, W1, b1, W2, b2, W3, b3, W4, b4):
    B = x.shape[0]
    bB = 2048
    while B % bB:
        bB //= 2
    args = (x, W0.T, b0.reshape(1, -1), W1.T, b1.reshape(1, -1),
            W2.T, b2.reshape(1, -1), W3.T, b3.reshape(1, -1),
            W4.reshape(1, -1), b4.reshape(1, 1),
            jnp.asarray(_ANC), jnp.asarray(_MASK))
    in_specs = [pl.BlockSpec((bB, x.shape[1]), lambda i: (i, 0))]
    for a in args[1:]:
        in_specs.append(pl.BlockSpec(a.shape, lambda i: (0, 0)))
    out_specs = [pl.BlockSpec((_N, bB), lambda i: (0, i))] * 2
    out_shape = [jax.ShapeDtypeStruct((_N, B), x.dtype)] * 2
    ppT, lgT = pl.pallas_call(
        _body, grid=(B // bB,), in_specs=in_specs,
        out_specs=out_specs, out_shape=out_shape)(*args)
    return ppT.T, lgT.T


# speedup vs baseline: 1.6783x; 1.0006x over previous
import numpy as np
import jax
import jax.numpy as jnp
from jax.experimental import pallas as pl

_PARENT = [-1, 0, 0, 0, 0, 1, 1, 2, 3, 4, 4, 5, 5, 6, 7, 8, 9, 10,
           11, 12, 13, 14, 14, 14, 15, 15, 16, 17]
_N = 28

_CHILD = [[] for _ in range(_N)]
for _c, _p in enumerate(_PARENT):
    if _p >= 0:
        _CHILD[_p].append(_c)

_deg = np.ones(_N, np.float64)
for _c, _p in enumerate(_PARENT):
    if _p >= 0:
        _deg[_c] += 1.0
        _deg[_p] += 1.0
_dinv = 1.0 / np.sqrt(_deg)
_A = np.zeros((_N, _N), np.float64)
for _i in range(_N):
    _A[_i, _i] = _dinv[_i] * _dinv[_i]
for _c, _p in enumerate(_PARENT):
    if _p >= 0:
        _A[_p, _c] = _dinv[_p] * _dinv[_c]
        _A[_c, _p] = _dinv[_c] * _dinv[_p]
_R = [float(v) for v in _A.sum(1)]
_ATERMS = [[(j, float(_A[i, j])) for j in range(_N) if _A[i, j] != 0.0]
           for i in range(_N)]

_ANC = np.zeros((_N, _N), np.float32)
for _i in range(1, _N):
    _n = _i
    while _n != 0:
        _ANC[_i, _n] = 1.0
        _n = _PARENT[_n]

_GROUPS = []
for _p in range(_N):
    _ch = _CHILD[_p]
    if len(_ch) > 1:
        assert _ch == list(range(_ch[0], _ch[0] + len(_ch)))
        _GROUPS.append((_ch[0], _ch[0] + len(_ch)))
_MASK = np.zeros((_N, 1), np.float32)
for _a, _b in _GROUPS:
    _MASK[_a:_b] = 1.0


def _body(x_ref, W0T_ref, b0_ref, W1T_ref, b1_ref, W2T_ref, b2_ref,
          W3T_ref, b3_ref, w4_ref, b4_ref, anc_ref, mask_ref,
          pp_ref, lg_ref):
    f32 = jnp.float32
    bB = x_ref.shape[0]
    xb = x_ref[...]
    y0T = jax.lax.dot_general(W0T_ref[...], xb, (((1,), (1,)), ((), ())),
                              preferred_element_type=f32)
    b0 = b0_ref[...].T
    h = jnp.concatenate(
        [jax.nn.relu(_R[i] * y0T + b0) for i in range(_N)], axis=1)

    for WT_ref, b_ref in ((W1T_ref, b1_ref), (W2T_ref, b2_ref),
                          (W3T_ref, b3_ref)):
        g = jnp.dot(WT_ref[...], h, preferred_element_type=f32)
        b = b_ref[...].T
        h = jnp.concatenate([
            jax.nn.relu(sum(c * g[:, j * bB:(j + 1) * bB]
                            for j, c in _ATERMS[i]) + b)
            for i in range(_N)], axis=1)

    w4 = w4_ref[...].T
    z = jnp.sum(h * w4, axis=0, keepdims=True)
    b4 = b4_ref[0, 0]
    lg = jnp.concatenate([
        sum(c * z[:, j * bB:(j + 1) * bB] for j, c in _ATERMS[i])
        for i in range(_N)], axis=0) + b4

    lses = []
    for a, bnd in _GROUPS:
        seg = lg[a:bnd]
        m = jnp.max(seg, axis=0, keepdims=True)
        lse = m + jnp.log(jnp.sum(jnp.exp(seg - m), axis=0, keepdims=True))
        lses.append(jnp.broadcast_to(lse, (bnd - a, bB)))
    zrow = jnp.zeros((1, bB), f32)
    lse_node = jnp.concatenate([
        zrow,
        lses[0],
        lses[1],
        jnp.broadcast_to(zrow, (2, bB)),
        lses[2],
        lses[3],
        jnp.broadcast_to(zrow, (8, bB)),
        lses[4],
        lses[5],
        jnp.broadcast_to(zrow, (2, bB)),
    ], axis=0)
    s = mask_ref[...] * lg - lse_node
    logp = jnp.dot(anc_ref[...], s, preferred_element_type=f32)
    pp_ref[...] = jnp.exp(logp)
    lg_ref[...] = lg


def kernel(x, W0, b0, W1, b1, W2, b2, W3, b3, W4, b4):
    B = x.shape[0]
    bB = 4096
    while B % bB:
        bB //= 2
    args = (x, W0.T, b0.reshape(1, -1), W1.T, b1.reshape(1, -1),
            W2.T, b2.reshape(1, -1), W3.T, b3.reshape(1, -1),
            W4.reshape(1, -1), b4.reshape(1, 1),
            jnp.asarray(_ANC), jnp.asarray(_MASK))
    in_specs = [pl.BlockSpec((bB, x.shape[1]), lambda i: (i, 0))]
    for a in args[1:]:
        in_specs.append(pl.BlockSpec(a.shape, lambda i: (0, 0)))
    out_specs = [pl.BlockSpec((_N, bB), lambda i: (0, i))] * 2
    out_shape = [jax.ShapeDtypeStruct((_N, B), x.dtype)] * 2
    ppT, lgT = pl.pallas_call(
        _body, grid=(B // bB,), in_specs=in_specs,
        out_specs=out_specs, out_shape=out_shape)(*args)
    return ppT.T, lgT.T


# speedup vs baseline: 1.7235x; 1.0269x over previous
import numpy as np
import jax
import jax.numpy as jnp
from jax.experimental import pallas as pl

_PARENT = [-1, 0, 0, 0, 0, 1, 1, 2, 3, 4, 4, 5, 5, 6, 7, 8, 9, 10,
           11, 12, 13, 14, 14, 14, 15, 15, 16, 17]
_N = 28

_CHILD = [[] for _ in range(_N)]
for _c, _p in enumerate(_PARENT):
    if _p >= 0:
        _CHILD[_p].append(_c)

_deg = np.ones(_N, np.float64)
for _c, _p in enumerate(_PARENT):
    if _p >= 0:
        _deg[_c] += 1.0
        _deg[_p] += 1.0
_dinv = 1.0 / np.sqrt(_deg)
_A = np.zeros((_N, _N), np.float64)
for _i in range(_N):
    _A[_i, _i] = _dinv[_i] * _dinv[_i]
for _c, _p in enumerate(_PARENT):
    if _p >= 0:
        _A[_p, _c] = _dinv[_p] * _dinv[_c]
        _A[_c, _p] = _dinv[_c] * _dinv[_p]
_R = [float(v) for v in _A.sum(1)]
_ATERMS = [[(j, float(_A[i, j])) for j in range(_N) if _A[i, j] != 0.0]
           for i in range(_N)]

_REPMAP = {10: 9, 17: 16, 27: 26, 12: 11, 19: 18, 22: 21, 23: 21, 25: 24}
_REP = [_REPMAP.get(i, i) for i in range(_N)]
_REPS = sorted(set(_REP))
_POS = {r: k for k, r in enumerate(_REPS)}
_NR = len(_REPS)
_RTERMS = []
for _i in _REPS:
    _acc = {}
    for _j, _c in _ATERMS[_i]:
        _k = _POS[_REP[_j]]
        _acc[_k] = _acc.get(_k, 0.0) + _c
    _RTERMS.append(sorted(_acc.items()))

_ANC = np.zeros((_N, _N), np.float32)
for _i in range(1, _N):
    _n = _i
    while _n != 0:
        _ANC[_i, _n] = 1.0
        _n = _PARENT[_n]

_GROUPS = []
for _p in range(_N):
    _ch = _CHILD[_p]
    if len(_ch) > 1:
        assert _ch == list(range(_ch[0], _ch[0] + len(_ch)))
        _GROUPS.append((_ch[0], _ch[0] + len(_ch)))
_MASK = np.zeros((_N, 1), np.float32)
for _a, _b in _GROUPS:
    _MASK[_a:_b] = 1.0


def _body(x_ref, W0T_ref, b0_ref, W1T_ref, b1_ref, W2T_ref, b2_ref,
          W3T_ref, b3_ref, w4_ref, b4_ref, anc_ref, mask_ref,
          pp_ref, lg_ref):
    f32 = jnp.float32
    bB = x_ref.shape[0]
    xb = x_ref[...]
    y0T = jax.lax.dot_general(W0T_ref[...], xb, (((1,), (1,)), ((), ())),
                              preferred_element_type=f32)
    b0 = b0_ref[...].T
    h = jnp.concatenate(
        [jax.nn.relu(_R[i] * y0T + b0) for i in _REPS], axis=1)

    for WT_ref, b_ref in ((W1T_ref, b1_ref), (W2T_ref, b2_ref),
                          (W3T_ref, b3_ref)):
        g = jnp.dot(WT_ref[...], h, preferred_element_type=f32)
        b = b_ref[...].T
        h = jnp.concatenate([
            jax.nn.relu(sum(c * g[:, k * bB:(k + 1) * bB]
                            for k, c in terms) + b)
            for terms in _RTERMS], axis=1)

    z = jnp.dot(w4_ref[...], h, preferred_element_type=f32)
    b4 = b4_ref[0, 0]
    lgr = [sum(c * z[:, k * bB:(k + 1) * bB] for k, c in terms)
           for terms in _RTERMS]
    lg = jnp.concatenate([lgr[_POS[_REP[i]]] for i in range(_N)],
                         axis=0) + b4

    lses = []
    for a, bnd in _GROUPS:
        seg = lg[a:bnd]
        m = jnp.max(seg, axis=0, keepdims=True)
        lse = m + jnp.log(jnp.sum(jnp.exp(seg - m), axis=0, keepdims=True))
        lses.append(jnp.broadcast_to(lse, (bnd - a, bB)))
    zrow = jnp.zeros((1, bB), f32)
    lse_node = jnp.concatenate([
        zrow,
        lses[0],
        lses[1],
        jnp.broadcast_to(zrow, (2, bB)),
        lses[2],
        lses[3],
        jnp.broadcast_to(zrow, (8, bB)),
        lses[4],
        lses[5],
        jnp.broadcast_to(zrow, (2, bB)),
    ], axis=0)
    s = mask_ref[...] * lg - lse_node
    logp = jnp.dot(anc_ref[...], s, preferred_element_type=f32)
    pp_ref[...] = jnp.exp(logp)
    lg_ref[...] = lg


def kernel(x, W0, b0, W1, b1, W2, b2, W3, b3, W4, b4):
    B = x.shape[0]
    bB = 4096
    while B % bB:
        bB //= 2
    args = (x, W0.T, b0.reshape(1, -1), W1.T, b1.reshape(1, -1),
            W2.T, b2.reshape(1, -1), W3.T, b3.reshape(1, -1),
            W4.reshape(1, -1), b4.reshape(1, 1),
            jnp.asarray(_ANC), jnp.asarray(_MASK))
    in_specs = [pl.BlockSpec((bB, x.shape[1]), lambda i: (i, 0))]
    for a in args[1:]:
        in_specs.append(pl.BlockSpec(a.shape, lambda i: (0, 0)))
    out_specs = [pl.BlockSpec((_N, bB), lambda i: (0, i))] * 2
    out_shape = [jax.ShapeDtypeStruct((_N, B), x.dtype)] * 2
    ppT, lgT = pl.pallas_call(
        _body, grid=(B // bB,), in_specs=in_specs,
        out_specs=out_specs, out_shape=out_shape)(*args)
    return ppT.T, lgT.T


# speedup vs baseline: 1.7471x; 1.0137x over previous
import numpy as np
import jax
import jax.numpy as jnp
from jax.experimental import pallas as pl

_PARENT = [-1, 0, 0, 0, 0, 1, 1, 2, 3, 4, 4, 5, 5, 6, 7, 8, 9, 10,
           11, 12, 13, 14, 14, 14, 15, 15, 16, 17]
_N = 28

_CHILD = [[] for _ in range(_N)]
for _c, _p in enumerate(_PARENT):
    if _p >= 0:
        _CHILD[_p].append(_c)

_deg = np.ones(_N, np.float64)
for _c, _p in enumerate(_PARENT):
    if _p >= 0:
        _deg[_c] += 1.0
        _deg[_p] += 1.0
_dinv = 1.0 / np.sqrt(_deg)
_A = np.zeros((_N, _N), np.float64)
for _i in range(_N):
    _A[_i, _i] = _dinv[_i] * _dinv[_i]
for _c, _p in enumerate(_PARENT):
    if _p >= 0:
        _A[_p, _c] = _dinv[_p] * _dinv[_c]
        _A[_c, _p] = _dinv[_c] * _dinv[_p]
_R = [float(v) for v in _A.sum(1)]
_ATERMS = [[(j, float(_A[i, j])) for j in range(_N) if _A[i, j] != 0.0]
           for i in range(_N)]

_REPMAP = {10: 9, 17: 16, 27: 26, 12: 11, 19: 18, 22: 21, 23: 21, 25: 24}
_REP = [_REPMAP.get(i, i) for i in range(_N)]
_REPS = sorted(set(_REP))
_POS = {r: k for k, r in enumerate(_REPS)}
_NR = len(_REPS)
_RTERMS = []
for _i in _REPS:
    _acc = {}
    for _j, _c in _ATERMS[_i]:
        _k = _POS[_REP[_j]]
        _acc[_k] = _acc.get(_k, 0.0) + _c
    _RTERMS.append(sorted(_acc.items()))

_ANC = np.zeros((_N, _N), np.float32)
for _i in range(1, _N):
    _n = _i
    while _n != 0:
        _ANC[_i, _n] = 1.0
        _n = _PARENT[_n]

_GROUPS = []
for _p in range(_N):
    _ch = _CHILD[_p]
    if len(_ch) > 1:
        assert _ch == list(range(_ch[0], _ch[0] + len(_ch)))
        _GROUPS.append((_ch[0], _ch[0] + len(_ch)))
_MASK = np.zeros((_N, 1), np.float32)
for _a, _b in _GROUPS:
    _MASK[_a:_b] = 1.0


def _body(x_ref, W0T_ref, b0_ref, W1T_ref, b1_ref, W2T_ref, b2_ref,
          W3T_ref, b3_ref, w4_ref, b4_ref, anc_ref, mask_ref,
          pp_ref, lg_ref):
    f32 = jnp.float32
    bB = x_ref.shape[0]
    xb = x_ref[...]
    y0T = jax.lax.dot_general(W0T_ref[...], xb, (((1,), (1,)), ((), ())),
                              preferred_element_type=f32)
    b0 = b0_ref[...].T
    h = jnp.concatenate(
        [jax.nn.relu(_R[i] * y0T + b0) for i in _REPS], axis=1)

    for WT_ref, b_ref in ((W1T_ref, b1_ref), (W2T_ref, b2_ref),
                          (W3T_ref, b3_ref)):
        g = jnp.dot(WT_ref[...], h, preferred_element_type=f32)
        b = b_ref[...].T
        h = jnp.concatenate([
            jax.nn.relu(sum(c * g[:, k * bB:(k + 1) * bB]
                            for k, c in terms) + b)
            for terms in _RTERMS], axis=1)

    z = jnp.dot(w4_ref[...], h, preferred_element_type=f32)
    b4 = b4_ref[0, 0]
    lgr = [sum(c * z[:, k * bB:(k + 1) * bB] for k, c in terms)
           for terms in _RTERMS]
    lg = jnp.concatenate([lgr[_POS[_REP[i]]] for i in range(_N)],
                         axis=0) + b4

    lses = []
    for a, bnd in _GROUPS:
        seg = lg[a:bnd]
        m = jnp.max(seg, axis=0, keepdims=True)
        lse = m + jnp.log(jnp.sum(jnp.exp(seg - m), axis=0, keepdims=True))
        lses.append(jnp.broadcast_to(lse, (bnd - a, bB)))
    zrow = jnp.zeros((1, bB), f32)
    lse_node = jnp.concatenate([
        zrow,
        lses[0],
        lses[1],
        jnp.broadcast_to(zrow, (2, bB)),
        lses[2],
        lses[3],
        jnp.broadcast_to(zrow, (8, bB)),
        lses[4],
        lses[5],
        jnp.broadcast_to(zrow, (2, bB)),
    ], axis=0)
    s = mask_ref[...] * lg - lse_node
    logp = jnp.dot(anc_ref[...], s, preferred_element_type=f32)
    pp_ref[...] = jnp.exp(logp)
    lg_ref[...] = lg


def kernel(x, W0, b0, W1, b1, W2, b2, W3, b3, W4, b4):
    B = x.shape[0]
    bB = 2048
    while B % bB:
        bB //= 2
    args = (x, W0.T, b0.reshape(1, -1), W1.T, b1.reshape(1, -1),
            W2.T, b2.reshape(1, -1), W3.T, b3.reshape(1, -1),
            W4.reshape(1, -1), b4.reshape(1, 1),
            jnp.asarray(_ANC), jnp.asarray(_MASK))
    in_specs = [pl.BlockSpec((bB, x.shape[1]), lambda i: (i, 0))]
    for a in args[1:]:
        in_specs.append(pl.BlockSpec(a.shape, lambda i: (0, 0)))
    out_specs = [pl.BlockSpec((_N, bB), lambda i: (0, i))] * 2
    out_shape = [jax.ShapeDtypeStruct((_N, B), x.dtype)] * 2
    ppT, lgT = pl.pallas_call(
        _body, grid=(B // bB,), in_specs=in_specs,
        out_specs=out_specs, out_shape=out_shape)(*args)
    return ppT.T, lgT.T
